# baseline (speedup 1.0000x reference)
import functools

import jax
import jax.numpy as jnp
from jax import lax
from jax.experimental import pallas as pl
from jax.experimental.pallas import tpu as pltpu
from jax.experimental.pallas import tpu_sc as plsc

_NC = 2
_NS = 16
_NW = _NC * _NS
_L = 16


def _knn_sc_body(BN, S, P, tpb, n_unroll,
                 qx_h, qy_h, qz_h, qq_h, c2x_h, c2y_h, c2z_h, cc_h,
                 w1_h, w2_h, w3_h, i1_h, i2_h, i3_h,
                 qxv, qyv, qzv, qqv, c2xv, c2yv, c2zv, ccv,
                 ow1, ow2, ow3, oi1, oi2, oi3):
    cid = lax.axis_index("c")
    sid = lax.axis_index("s")
    wid = sid * _NC + cid
    base = wid * P
    b = wid // tpb

    pltpu.sync_copy(qx_h.at[pl.ds(base, P)], qxv)
    pltpu.sync_copy(qy_h.at[pl.ds(base, P)], qyv)
    pltpu.sync_copy(qz_h.at[pl.ds(base, P)], qzv)
    pltpu.sync_copy(qq_h.at[pl.ds(base, P)], qqv)
    pltpu.sync_copy(c2x_h.at[b], c2xv)
    pltpu.sync_copy(c2y_h.at[b], c2yv)
    pltpu.sync_copy(c2z_h.at[b], c2zv)
    pltpu.sync_copy(cc_h.at[b], ccv)

    inf = jnp.float32(jnp.inf)

    def group(g, _):
        qx = qxv[pl.ds(g * _L, _L)]
        qy = qyv[pl.ds(g * _L, _L)]
        qz = qzv[pl.ds(g * _L, _L)]
        qq = qqv[pl.ds(g * _L, _L)]

        def chunk(k, carry):
            d1, d2, d3, i1, i2, i3 = carry
            s0 = k * _L
            for j in range(_L):
                csl = pl.ds((s0 + j) * _L, _L)
                cx = c2xv[csl]
                cy = c2yv[csl]
                cz = c2zv[csl]
                cs = ccv[csl]
                dist = (qq - ((qx * cx + qy * cy) + qz * cz)) + cs
                sv = jnp.full((_L,), s0 + j, dtype=jnp.int32)
                c1 = dist < d1
                t1 = jnp.maximum(d1, dist)
                d1 = jnp.minimum(d1, dist)
                ti1 = jnp.where(c1, i1, sv)
                i1 = jnp.where(c1, sv, i1)
                c2 = t1 < d2
                t2 = jnp.maximum(d2, t1)
                d2 = jnp.minimum(d2, t1)
                ti2 = jnp.where(c2, i2, ti1)
                i2 = jnp.where(c2, ti1, i2)
                c3 = t2 < d3
                d3 = jnp.minimum(d3, t2)
                i3 = jnp.where(c3, ti2, i3)
            return d1, d2, d3, i1, i2, i3

        zi = jnp.zeros((_L,), jnp.int32)
        fi = jnp.full((_L,), inf, jnp.float32)
        d1, d2, d3, i1, i2, i3 = lax.fori_loop(
            0, S // _L, chunk, (fi, fi, fi, zi, zi, zi), unroll=n_unroll)

        r1 = 1.0 / (d1 + 1e-8)
        r2 = 1.0 / (d2 + 1e-8)
        r3 = 1.0 / (d3 + 1e-8)
        norm = r1 + r2 + r3
        sl = pl.ds(g * _L, _L)
        ow1[sl] = r1 / norm
        ow2[sl] = r2 / norm
        ow3[sl] = r3 / norm
        oi1[sl] = i1
        oi2[sl] = i2
        oi3[sl] = i3
        return 0

    lax.fori_loop(0, P // _L, group, 0)

    pltpu.sync_copy(ow1, w1_h.at[pl.ds(base, P)])
    pltpu.sync_copy(ow2, w2_h.at[pl.ds(base, P)])
    pltpu.sync_copy(ow3, w3_h.at[pl.ds(base, P)])
    pltpu.sync_copy(oi1, i1_h.at[pl.ds(base, P)])
    pltpu.sync_copy(oi2, i2_h.at[pl.ds(base, P)])
    pltpu.sync_copy(oi3, i3_h.at[pl.ds(base, P)])


def _knn_sc(qxyz, cxyz, n_unroll=2):
    B, N, _ = qxyz.shape
    S = cxyz.shape[1]
    BN = B * N
    P = BN // _NW
    tpb = _NW // B

    def _rnbf16(x):
        u = jax.lax.bitcast_convert_type(x, jnp.uint32)
        lsb = (u >> 16) & jnp.uint32(1)
        u = (u + jnp.uint32(0x7FFF) + lsb) & jnp.uint32(0xFFFF0000)
        return jax.lax.bitcast_convert_type(u, jnp.float32)

    qb = _rnbf16(qxyz)
    cb = _rnbf16(cxyz)
    qx = qb[..., 0].reshape(BN)
    qy = qb[..., 1].reshape(BN)
    qz = qb[..., 2].reshape(BN)
    qq = jnp.sum(qxyz * qxyz, axis=-1).reshape(BN)
    def _rep(a):
        return jnp.repeat(a[:, :, None], _L, axis=2).reshape(B, S * _L)

    c2x = _rep(2.0 * cb[..., 0])
    c2y = _rep(2.0 * cb[..., 1])
    c2z = _rep(2.0 * cb[..., 2])
    cc = _rep(jnp.sum(cxyz * cxyz, axis=-1))

    mesh = plsc.VectorSubcoreMesh(core_axis_name="c", subcore_axis_name="s")
    f32 = jnp.float32
    i32 = jnp.int32
    body = functools.partial(_knn_sc_body, BN, S, P, tpb, n_unroll)
    out = pl.kernel(
        body,
        out_type=[
            jax.ShapeDtypeStruct((BN,), f32),
            jax.ShapeDtypeStruct((BN,), f32),
            jax.ShapeDtypeStruct((BN,), f32),
            jax.ShapeDtypeStruct((BN,), i32),
            jax.ShapeDtypeStruct((BN,), i32),
            jax.ShapeDtypeStruct((BN,), i32),
        ],
        mesh=mesh,
        scratch_types=[
            pltpu.VMEM((P,), f32), pltpu.VMEM((P,), f32), pltpu.VMEM((P,), f32),
            pltpu.VMEM((P,), f32),
            pltpu.VMEM((S * _L,), f32), pltpu.VMEM((S * _L,), f32),
            pltpu.VMEM((S * _L,), f32), pltpu.VMEM((S * _L,), f32),
            pltpu.VMEM((P,), f32), pltpu.VMEM((P,), f32), pltpu.VMEM((P,), f32),
            pltpu.VMEM((P,), i32), pltpu.VMEM((P,), i32), pltpu.VMEM((P,), i32),
        ],
    )(qx, qy, qz, qq, c2x, c2y, c2z, cc)
    return out


def _compose_sc_body(N1, P, tpb, *refs):
    w1_h = refs[0:3]
    i1_h = refs[3:6]
    w2_h = refs[6:9]
    i2_h = refs[9:12]
    wo_h = refs[12:21]
    io_h = refs[21:30]
    w2v = refs[30:33]
    i2v = refs[33:36]
    wov = refs[36:45]
    iov = refs[45:54]
    w1g = refs[54:57]
    iadj = refs[57]
    sem = refs[58]

    cid = lax.axis_index("c")
    sid = lax.axis_index("s")
    wid = sid * _NC + cid
    base = wid * P
    b = wid // tpb

    for t in range(3):
        pltpu.sync_copy(w2_h[t].at[pl.ds(base, P)], w2v[t])
        pltpu.sync_copy(i2_h[t].at[pl.ds(base, P)], i2v[t])

    for k in range(3):
        def adj(g, _):
            sl = pl.ds(g * _L, _L)
            iadj[sl] = i2v[k][sl] + b * N1
            return 0
        lax.fori_loop(0, P // _L, adj, 0)
        for j in range(3):
            pltpu.async_copy(i1_h[j].at[iadj], iov[3 * k + j], sem).wait()
            pltpu.async_copy(w1_h[j].at[iadj], w1g[j], sem).wait()

        def mul(g, _):
            sl = pl.ds(g * _L, _L)
            w2 = w2v[k][sl]
            for j in range(3):
                wov[3 * k + j][sl] = w2 * w1g[j][sl]
            return 0
        lax.fori_loop(0, P // _L, mul, 0)

    for t in range(9):
        pltpu.sync_copy(wov[t], wo_h[t].at[pl.ds(base, P)])
        pltpu.sync_copy(iov[t], io_h[t].at[pl.ds(base, P)])


def _compose_sc(knn1, knn2, B, N1, BN2):
    P = BN2 // _NW
    tpb = _NW // B
    f32, i32 = jnp.float32, jnp.int32
    mesh = plsc.VectorSubcoreMesh(core_axis_name="c", subcore_axis_name="s")
    w1a, w1b, w1c, i1a, i1b, i1c = knn1
    w2a, w2b, w2c, i2a, i2b, i2c = knn2
    body = functools.partial(_compose_sc_body, N1, P, tpb)
    outs = pl.kernel(
        body,
        out_type=[jax.ShapeDtypeStruct((BN2,), f32)] * 9
                 + [jax.ShapeDtypeStruct((BN2,), i32)] * 9,
        mesh=mesh,
        scratch_types=[pltpu.VMEM((P,), f32)] * 3
                      + [pltpu.VMEM((P,), i32)] * 3
                      + [pltpu.VMEM((P,), f32)] * 9
                      + [pltpu.VMEM((P,), i32)] * 9
                      + [pltpu.VMEM((P,), f32)] * 3
                      + [pltpu.VMEM((P,), i32)]
                      + [pltpu.SemaphoreType.DMA],
    )(w1a, w1b, w1c, i1a, i1b, i1c, w2a, w2b, w2c, i2a, i2b, i2c)
    return outs[:9], outs[9:]


def _interp_body(D1, S, p1_ref, p2_ref, w1_r, w2_r, w3_r, i1_r, i2_r, i3_r,
                 o_ref):
    n_tile = w1_r.shape[-1]
    iota = lax.broadcasted_iota(jnp.int32, (n_tile, S), 1)
    i1 = i1_r[0, 0, :]
    i2 = i2_r[0, 0, :]
    i3 = i3_r[0, 0, :]
    w1 = w1_r[0, 0, :]
    w2 = w2_r[0, 0, :]
    w3 = w3_r[0, 0, :]
    w = (jnp.where(iota == i1[:, None], w1[:, None], 0.0)
         + jnp.where(iota == i2[:, None], w2[:, None], 0.0)
         + jnp.where(iota == i3[:, None], w3[:, None], 0.0))
    interp = lax.dot_general(p2_ref[0], w, (((1,), (1,)), ((), ())),
                             preferred_element_type=jnp.float32)
    o_ref[0, :D1, :] = p1_ref[0]
    o_ref[0, D1:, :] = interp


def _interp_tc(points1, points2, knn, n_tile):
    B, D1, N = points1.shape
    S = points2.shape[2]
    D2 = points2.shape[1]
    NT = N // n_tile
    w1, w2, w3, i1, i2, i3 = (a.reshape(B * NT, 1, n_tile) for a in knn)

    body = functools.partial(_interp_body, D1, S)
    knn_spec = pl.BlockSpec((1, 1, n_tile), lambda b, n: (b * NT + n, 0, 0))
    return pl.pallas_call(
        body,
        grid=(B, NT),
        in_specs=[
            pl.BlockSpec((1, D1, n_tile), lambda b, n: (b, 0, n)),
            pl.BlockSpec((1, D2, S), lambda b, n: (b, 0, 0)),
            knn_spec, knn_spec, knn_spec, knn_spec, knn_spec, knn_spec,
        ],
        out_specs=pl.BlockSpec((1, D1 + D2, n_tile), lambda b, n: (b, 0, n)),
        out_shape=jax.ShapeDtypeStruct((B, D1 + D2, N), jnp.float32),
    )(points1, points2, w1, w2, w3, i1, i2, i3)


def _onehot_w(iota, idx_ws):
    w = None
    for i, wv in idx_ws:
        term = jnp.where(iota == i[:, None], wv[:, None], 0.0)
        w = term if w is None else w + term
    return w


def _fused_body(D0, S1, S2, x0_ref, x1_ref, x2_ref,
                w2a_r, w2b_r, w2c_r, i2a_r, i2b_r, i2c_r,
                *comp_refs):
    o_ref = comp_refs[-1]
    wo_rs = comp_refs[:9]
    io_rs = comp_refs[9:18]
    n_tile = w2a_r.shape[-1]
    D1 = x1_ref.shape[1]
    D2 = x2_ref.shape[1]
    iota2 = lax.broadcasted_iota(jnp.int32, (n_tile, S2), 1)
    w2 = _onehot_w(iota2, [(i2a_r[0, 0, :], w2a_r[0, 0, :]),
                           (i2b_r[0, 0, :], w2b_r[0, 0, :]),
                           (i2c_r[0, 0, :], w2c_r[0, 0, :])])
    mid = lax.dot_general(x1_ref[0], w2, (((1,), (1,)), ((), ())),
                          preferred_element_type=jnp.float32)
    iota1 = lax.broadcasted_iota(jnp.int32, (n_tile, S1), 1)
    w21 = _onehot_w(iota1, [(io_rs[t][0, 0, :], wo_rs[t][0, 0, :])
                            for t in range(9)])
    low = lax.dot_general(x2_ref[0], w21, (((1,), (1,)), ((), ())),
                          preferred_element_type=jnp.float32)
    o_ref[0, :D0, :] = x0_ref[0]
    o_ref[0, D0:D0 + D1, :] = mid
    o_ref[0, D0 + D1:, :] = low


def _interp_fused_tc(x0, x1, x2, knn2, wo, io, n_tile):
    B, D0, N = x0.shape
    D1, S2 = x1.shape[1], x1.shape[2]
    D2, S1 = x2.shape[1], x2.shape[2]
    NT = N // n_tile
    k2 = [a.reshape(B * NT, 1, n_tile) for a in knn2]
    wos = [wo[t].reshape(B * NT, 1, n_tile) for t in range(9)]
    ios = [io[t].reshape(B * NT, 1, n_tile) for t in range(9)]

    body = functools.partial(_fused_body, D0, S1, S2)
    knn_spec = pl.BlockSpec((1, 1, n_tile), lambda b, n: (b * NT + n, 0, 0))
    return pl.pallas_call(
        body,
        grid=(B, NT),
        in_specs=[
            pl.BlockSpec((1, D0, n_tile), lambda b, n: (b, 0, n)),
            pl.BlockSpec((1, D1, S2), lambda b, n: (b, 0, 0)),
            pl.BlockSpec((1, D2, S1), lambda b, n: (b, 0, 0)),
        ] + [knn_spec] * 24,
        out_specs=pl.BlockSpec((1, D0 + D1 + D2, n_tile), lambda b, n: (b, 0, n)),
        out_shape=jax.ShapeDtypeStruct((B, D0 + D1 + D2, N), jnp.float32),
    )(x0, x1, x2, *k2, *wos, *ios)


def kernel(xyz0, xyz1, xyz2, x0, x1, x2):
    B, N1 = xyz1.shape[0], xyz1.shape[1]
    N2 = xyz0.shape[1]
    knn1 = _knn_sc(xyz1, xyz2)
    knn2 = _knn_sc(xyz0, xyz1)
    wo, io = _compose_sc(knn1, knn2, B, N1, B * N2)
    out = _interp_fused_tc(x0, x1, x2, knn2, wo, io, 256)
    return out


# speedup vs baseline: 1.0250x; 1.0250x over previous
import functools

import jax
import jax.numpy as jnp
from jax import lax
from jax.experimental import pallas as pl
from jax.experimental.pallas import tpu as pltpu
from jax.experimental.pallas import tpu_sc as plsc

_NC = 2
_NS = 16
_NW = _NC * _NS
_L = 16


def _knn_sc_body(BN, S, P, tpb, n_unroll,
                 qx_h, qy_h, qz_h, qq_h, c2x_h, c2y_h, c2z_h, cc_h,
                 w1_h, w2_h, w3_h, i1_h, i2_h, i3_h,
                 qxv, qyv, qzv, qqv, c2xv, c2yv, c2zv, ccv,
                 ow1, ow2, ow3, oi1, oi2, oi3):
    cid = lax.axis_index("c")
    sid = lax.axis_index("s")
    wid = sid * _NC + cid
    base = wid * P
    b = wid // tpb

    pltpu.sync_copy(qx_h.at[pl.ds(base, P)], qxv)
    pltpu.sync_copy(qy_h.at[pl.ds(base, P)], qyv)
    pltpu.sync_copy(qz_h.at[pl.ds(base, P)], qzv)
    pltpu.sync_copy(qq_h.at[pl.ds(base, P)], qqv)
    pltpu.sync_copy(c2x_h.at[b], c2xv)
    pltpu.sync_copy(c2y_h.at[b], c2yv)
    pltpu.sync_copy(c2z_h.at[b], c2zv)
    pltpu.sync_copy(cc_h.at[b], ccv)

    inf = jnp.float32(jnp.inf)

    def group(g, _):
        qx = qxv[pl.ds(g * _L, _L)]
        qy = qyv[pl.ds(g * _L, _L)]
        qz = qzv[pl.ds(g * _L, _L)]
        qq = qqv[pl.ds(g * _L, _L)]

        def chunk(k, carry):
            d1, d2, d3, i1, i2, i3 = carry
            s0 = k * _L
            for j in range(_L):
                csl = pl.ds((s0 + j) * _L, _L)
                cx = c2xv[csl]
                cy = c2yv[csl]
                cz = c2zv[csl]
                cs = ccv[csl]
                dist = (qq - ((qx * cx + qy * cy) + qz * cz)) + cs
                sv = jnp.full((_L,), s0 + j, dtype=jnp.int32)
                c1 = dist < d1
                t1 = jnp.maximum(d1, dist)
                d1 = jnp.minimum(d1, dist)
                ti1 = jnp.where(c1, i1, sv)
                i1 = jnp.where(c1, sv, i1)
                c2 = t1 < d2
                t2 = jnp.maximum(d2, t1)
                d2 = jnp.minimum(d2, t1)
                ti2 = jnp.where(c2, i2, ti1)
                i2 = jnp.where(c2, ti1, i2)
                c3 = t2 < d3
                d3 = jnp.minimum(d3, t2)
                i3 = jnp.where(c3, ti2, i3)
            return d1, d2, d3, i1, i2, i3

        zi = jnp.zeros((_L,), jnp.int32)
        fi = jnp.full((_L,), inf, jnp.float32)
        d1, d2, d3, i1, i2, i3 = lax.fori_loop(
            0, S // _L, chunk, (fi, fi, fi, zi, zi, zi), unroll=n_unroll)

        r1 = 1.0 / (d1 + 1e-8)
        r2 = 1.0 / (d2 + 1e-8)
        r3 = 1.0 / (d3 + 1e-8)
        norm = r1 + r2 + r3
        sl = pl.ds(g * _L, _L)
        ow1[sl] = r1 / norm
        ow2[sl] = r2 / norm
        ow3[sl] = r3 / norm
        oi1[sl] = i1
        oi2[sl] = i2
        oi3[sl] = i3
        return 0

    lax.fori_loop(0, P // _L, group, 0)

    pltpu.sync_copy(ow1, w1_h.at[pl.ds(base, P)])
    pltpu.sync_copy(ow2, w2_h.at[pl.ds(base, P)])
    pltpu.sync_copy(ow3, w3_h.at[pl.ds(base, P)])
    pltpu.sync_copy(oi1, i1_h.at[pl.ds(base, P)])
    pltpu.sync_copy(oi2, i2_h.at[pl.ds(base, P)])
    pltpu.sync_copy(oi3, i3_h.at[pl.ds(base, P)])


def _knn_sc(qxyz, cxyz, n_unroll=2):
    B, N, _ = qxyz.shape
    S = cxyz.shape[1]
    BN = B * N
    P = BN // _NW
    tpb = _NW // B

    def _rnbf16(x):
        u = jax.lax.bitcast_convert_type(x, jnp.uint32)
        lsb = (u >> 16) & jnp.uint32(1)
        u = (u + jnp.uint32(0x7FFF) + lsb) & jnp.uint32(0xFFFF0000)
        return jax.lax.bitcast_convert_type(u, jnp.float32)

    qb = _rnbf16(qxyz)
    cb = _rnbf16(cxyz)
    qx = qb[..., 0].reshape(BN)
    qy = qb[..., 1].reshape(BN)
    qz = qb[..., 2].reshape(BN)
    qq = jnp.sum(qxyz * qxyz, axis=-1).reshape(BN)
    def _rep(a):
        return jnp.repeat(a[:, :, None], _L, axis=2).reshape(B, S * _L)

    c2x = _rep(2.0 * cb[..., 0])
    c2y = _rep(2.0 * cb[..., 1])
    c2z = _rep(2.0 * cb[..., 2])
    cc = _rep(jnp.sum(cxyz * cxyz, axis=-1))

    mesh = plsc.VectorSubcoreMesh(core_axis_name="c", subcore_axis_name="s")
    f32 = jnp.float32
    i32 = jnp.int32
    body = functools.partial(_knn_sc_body, BN, S, P, tpb, n_unroll)
    out = pl.kernel(
        body,
        out_type=[
            jax.ShapeDtypeStruct((BN,), f32),
            jax.ShapeDtypeStruct((BN,), f32),
            jax.ShapeDtypeStruct((BN,), f32),
            jax.ShapeDtypeStruct((BN,), i32),
            jax.ShapeDtypeStruct((BN,), i32),
            jax.ShapeDtypeStruct((BN,), i32),
        ],
        mesh=mesh,
        scratch_types=[
            pltpu.VMEM((P,), f32), pltpu.VMEM((P,), f32), pltpu.VMEM((P,), f32),
            pltpu.VMEM((P,), f32),
            pltpu.VMEM((S * _L,), f32), pltpu.VMEM((S * _L,), f32),
            pltpu.VMEM((S * _L,), f32), pltpu.VMEM((S * _L,), f32),
            pltpu.VMEM((P,), f32), pltpu.VMEM((P,), f32), pltpu.VMEM((P,), f32),
            pltpu.VMEM((P,), i32), pltpu.VMEM((P,), i32), pltpu.VMEM((P,), i32),
        ],
    )(qx, qy, qz, qq, c2x, c2y, c2z, cc)
    return out


def _compose_sc_body(N1, P, tpb, *refs):
    w1_h = refs[0:3]
    i1_h = refs[3:6]
    w2_h = refs[6:9]
    i2_h = refs[9:12]
    wo_h = refs[12:21]
    io_h = refs[21:30]
    w2v = refs[30:33]
    i2v = refs[33:36]
    wov = refs[36:45]
    iov = refs[45:54]
    w1g = refs[54:63]
    iadj = refs[63:66]
    sem = refs[66]

    cid = lax.axis_index("c")
    sid = lax.axis_index("s")
    wid = sid * _NC + cid
    base = wid * P
    b = wid // tpb

    for t in range(3):
        pltpu.sync_copy(w2_h[t].at[pl.ds(base, P)], w2v[t])
        pltpu.sync_copy(i2_h[t].at[pl.ds(base, P)], i2v[t])

    def adj(g, _):
        sl = pl.ds(g * _L, _L)
        for k in range(3):
            iadj[k][sl] = i2v[k][sl] + b * N1
        return 0
    lax.fori_loop(0, P // _L, adj, 0, unroll=4)

    copies = []
    for k in range(3):
        for j in range(3):
            copies.append(pltpu.make_async_copy(
                i1_h[j].at[iadj[k]], iov[3 * k + j], sem))
            copies.append(pltpu.make_async_copy(
                w1_h[j].at[iadj[k]], w1g[3 * k + j], sem))
    for c in copies:
        c.start()
    for c in copies:
        c.wait()

    def mul(g, _):
        sl = pl.ds(g * _L, _L)
        for k in range(3):
            w2 = w2v[k][sl]
            for j in range(3):
                wov[3 * k + j][sl] = w2 * w1g[3 * k + j][sl]
        return 0
    lax.fori_loop(0, P // _L, mul, 0, unroll=4)

    for t in range(9):
        pltpu.sync_copy(wov[t], wo_h[t].at[pl.ds(base, P)])
        pltpu.sync_copy(iov[t], io_h[t].at[pl.ds(base, P)])


def _compose_sc(knn1, knn2, B, N1, BN2):
    P = BN2 // _NW
    tpb = _NW // B
    f32, i32 = jnp.float32, jnp.int32
    mesh = plsc.VectorSubcoreMesh(core_axis_name="c", subcore_axis_name="s")
    w1a, w1b, w1c, i1a, i1b, i1c = knn1
    w2a, w2b, w2c, i2a, i2b, i2c = knn2
    body = functools.partial(_compose_sc_body, N1, P, tpb)
    outs = pl.kernel(
        body,
        out_type=[jax.ShapeDtypeStruct((BN2,), f32)] * 9
                 + [jax.ShapeDtypeStruct((BN2,), i32)] * 9,
        mesh=mesh,
        scratch_types=[pltpu.VMEM((P,), f32)] * 3
                      + [pltpu.VMEM((P,), i32)] * 3
                      + [pltpu.VMEM((P,), f32)] * 9
                      + [pltpu.VMEM((P,), i32)] * 9
                      + [pltpu.VMEM((P,), f32)] * 9
                      + [pltpu.VMEM((P,), i32)] * 3
                      + [pltpu.SemaphoreType.DMA],
    )(w1a, w1b, w1c, i1a, i1b, i1c, w2a, w2b, w2c, i2a, i2b, i2c)
    return outs[:9], outs[9:]


def _interp_body(D1, S, p1_ref, p2_ref, w1_r, w2_r, w3_r, i1_r, i2_r, i3_r,
                 o_ref):
    n_tile = w1_r.shape[-1]
    iota = lax.broadcasted_iota(jnp.int32, (n_tile, S), 1)
    i1 = i1_r[0, 0, :]
    i2 = i2_r[0, 0, :]
    i3 = i3_r[0, 0, :]
    w1 = w1_r[0, 0, :]
    w2 = w2_r[0, 0, :]
    w3 = w3_r[0, 0, :]
    w = (jnp.where(iota == i1[:, None], w1[:, None], 0.0)
         + jnp.where(iota == i2[:, None], w2[:, None], 0.0)
         + jnp.where(iota == i3[:, None], w3[:, None], 0.0))
    interp = lax.dot_general(p2_ref[0], w, (((1,), (1,)), ((), ())),
                             preferred_element_type=jnp.float32)
    o_ref[0, :D1, :] = p1_ref[0]
    o_ref[0, D1:, :] = interp


def _interp_tc(points1, points2, knn, n_tile):
    B, D1, N = points1.shape
    S = points2.shape[2]
    D2 = points2.shape[1]
    NT = N // n_tile
    w1, w2, w3, i1, i2, i3 = (a.reshape(B * NT, 1, n_tile) for a in knn)

    body = functools.partial(_interp_body, D1, S)
    knn_spec = pl.BlockSpec((1, 1, n_tile), lambda b, n: (b * NT + n, 0, 0))
    return pl.pallas_call(
        body,
        grid=(B, NT),
        in_specs=[
            pl.BlockSpec((1, D1, n_tile), lambda b, n: (b, 0, n)),
            pl.BlockSpec((1, D2, S), lambda b, n: (b, 0, 0)),
            knn_spec, knn_spec, knn_spec, knn_spec, knn_spec, knn_spec,
        ],
        out_specs=pl.BlockSpec((1, D1 + D2, n_tile), lambda b, n: (b, 0, n)),
        out_shape=jax.ShapeDtypeStruct((B, D1 + D2, N), jnp.float32),
    )(points1, points2, w1, w2, w3, i1, i2, i3)


def _onehot_w(iota, idx_ws):
    w = None
    for i, wv in idx_ws:
        term = jnp.where(iota == i[:, None], wv[:, None], 0.0)
        w = term if w is None else w + term
    return w


def _fused_body(D0, S1, S2, x0_ref, x1_ref, x2_ref,
                w2a_r, w2b_r, w2c_r, i2a_r, i2b_r, i2c_r,
                *comp_refs):
    o_ref = comp_refs[-1]
    wo_rs = comp_refs[:9]
    io_rs = comp_refs[9:18]
    n_tile = w2a_r.shape[-1]
    D1 = x1_ref.shape[1]
    D2 = x2_ref.shape[1]
    iota2 = lax.broadcasted_iota(jnp.int32, (n_tile, S2), 1)
    w2 = _onehot_w(iota2, [(i2a_r[0, 0, :], w2a_r[0, 0, :]),
                           (i2b_r[0, 0, :], w2b_r[0, 0, :]),
                           (i2c_r[0, 0, :], w2c_r[0, 0, :])])
    mid = lax.dot_general(x1_ref[0], w2, (((1,), (1,)), ((), ())),
                          preferred_element_type=jnp.float32)
    iota1 = lax.broadcasted_iota(jnp.int32, (n_tile, S1), 1)
    w21 = _onehot_w(iota1, [(io_rs[t][0, 0, :], wo_rs[t][0, 0, :])
                            for t in range(9)])
    low = lax.dot_general(x2_ref[0], w21, (((1,), (1,)), ((), ())),
                          preferred_element_type=jnp.float32)
    o_ref[0, :D0, :] = x0_ref[0]
    o_ref[0, D0:D0 + D1, :] = mid
    o_ref[0, D0 + D1:, :] = low


def _interp_fused_tc(x0, x1, x2, knn2, wo, io, n_tile):
    B, D0, N = x0.shape
    D1, S2 = x1.shape[1], x1.shape[2]
    D2, S1 = x2.shape[1], x2.shape[2]
    NT = N // n_tile
    k2 = [a.reshape(B * NT, 1, n_tile) for a in knn2]
    wos = [wo[t].reshape(B * NT, 1, n_tile) for t in range(9)]
    ios = [io[t].reshape(B * NT, 1, n_tile) for t in range(9)]

    body = functools.partial(_fused_body, D0, S1, S2)
    knn_spec = pl.BlockSpec((1, 1, n_tile), lambda b, n: (b * NT + n, 0, 0))
    return pl.pallas_call(
        body,
        grid=(B, NT),
        in_specs=[
            pl.BlockSpec((1, D0, n_tile), lambda b, n: (b, 0, n)),
            pl.BlockSpec((1, D1, S2), lambda b, n: (b, 0, 0)),
            pl.BlockSpec((1, D2, S1), lambda b, n: (b, 0, 0)),
        ] + [knn_spec] * 24,
        out_specs=pl.BlockSpec((1, D0 + D1 + D2, n_tile), lambda b, n: (b, 0, n)),
        out_shape=jax.ShapeDtypeStruct((B, D0 + D1 + D2, N), jnp.float32),
    )(x0, x1, x2, *k2, *wos, *ios)


def kernel(xyz0, xyz1, xyz2, x0, x1, x2):
    B, N1 = xyz1.shape[0], xyz1.shape[1]
    N2 = xyz0.shape[1]
    knn1 = _knn_sc(xyz1, xyz2)
    knn2 = _knn_sc(xyz0, xyz1)
    wo, io = _compose_sc(knn1, knn2, B, N1, B * N2)
    out = _interp_fused_tc(x0, x1, x2, knn2, wo, io, 256)
    return out


# speedup vs baseline: 1.2185x; 1.1888x over previous
import functools

import jax
import jax.numpy as jnp
from jax import lax
from jax.experimental import pallas as pl
from jax.experimental.pallas import tpu as pltpu
from jax.experimental.pallas import tpu_sc as plsc

_NC = 2
_NS = 16
_NW = _NC * _NS
_L = 16


def _knn_sc_body(BN, S, P, tpb, n_unroll,
                 qx_h, qy_h, qz_h, qq_h, c2x_h, c2y_h, c2z_h, cc_h,
                 w1_h, w2_h, w3_h, i1_h, i2_h, i3_h,
                 qxv, qyv, qzv, qqv, c2xv, c2yv, c2zv, ccv,
                 ow1, ow2, ow3, oi1, oi2, oi3):
    cid = lax.axis_index("c")
    sid = lax.axis_index("s")
    wid = sid * _NC + cid
    base = wid * P
    b = wid // tpb

    pltpu.sync_copy(qx_h.at[pl.ds(base, P)], qxv)
    pltpu.sync_copy(qy_h.at[pl.ds(base, P)], qyv)
    pltpu.sync_copy(qz_h.at[pl.ds(base, P)], qzv)
    pltpu.sync_copy(qq_h.at[pl.ds(base, P)], qqv)
    pltpu.sync_copy(c2x_h.at[b], c2xv)
    pltpu.sync_copy(c2y_h.at[b], c2yv)
    pltpu.sync_copy(c2z_h.at[b], c2zv)
    pltpu.sync_copy(cc_h.at[b], ccv)

    inf = jnp.float32(jnp.inf)

    def group(g, _):
        qx = qxv[pl.ds(g * _L, _L)]
        qy = qyv[pl.ds(g * _L, _L)]
        qz = qzv[pl.ds(g * _L, _L)]
        qq = qqv[pl.ds(g * _L, _L)]

        def chunk(k, carry):
            d1, d2, d3, i1, i2, i3 = carry
            s0 = k * _L
            for j in range(_L):
                csl = pl.ds((s0 + j) * _L, _L)
                cx = c2xv[csl]
                cy = c2yv[csl]
                cz = c2zv[csl]
                cs = ccv[csl]
                dist = (qq - ((qx * cx + qy * cy) + qz * cz)) + cs
                sv = jnp.full((_L,), s0 + j, dtype=jnp.int32)
                c1 = dist < d1
                t1 = jnp.maximum(d1, dist)
                d1 = jnp.minimum(d1, dist)
                ti1 = jnp.where(c1, i1, sv)
                i1 = jnp.where(c1, sv, i1)
                c2 = t1 < d2
                t2 = jnp.maximum(d2, t1)
                d2 = jnp.minimum(d2, t1)
                ti2 = jnp.where(c2, i2, ti1)
                i2 = jnp.where(c2, ti1, i2)
                c3 = t2 < d3
                d3 = jnp.minimum(d3, t2)
                i3 = jnp.where(c3, ti2, i3)
            return d1, d2, d3, i1, i2, i3

        zi = jnp.zeros((_L,), jnp.int32)
        fi = jnp.full((_L,), inf, jnp.float32)
        d1, d2, d3, i1, i2, i3 = lax.fori_loop(
            0, S // _L, chunk, (fi, fi, fi, zi, zi, zi), unroll=n_unroll)

        r1 = 1.0 / (d1 + 1e-8)
        r2 = 1.0 / (d2 + 1e-8)
        r3 = 1.0 / (d3 + 1e-8)
        norm = r1 + r2 + r3
        sl = pl.ds(g * _L, _L)
        ow1[sl] = r1 / norm
        ow2[sl] = r2 / norm
        ow3[sl] = r3 / norm
        oi1[sl] = i1
        oi2[sl] = i2
        oi3[sl] = i3
        return 0

    lax.fori_loop(0, P // _L, group, 0)

    pltpu.sync_copy(ow1, w1_h.at[pl.ds(base, P)])
    pltpu.sync_copy(ow2, w2_h.at[pl.ds(base, P)])
    pltpu.sync_copy(ow3, w3_h.at[pl.ds(base, P)])
    pltpu.sync_copy(oi1, i1_h.at[pl.ds(base, P)])
    pltpu.sync_copy(oi2, i2_h.at[pl.ds(base, P)])
    pltpu.sync_copy(oi3, i3_h.at[pl.ds(base, P)])


def _knn_sc(qxyz, cxyz, n_unroll=4):
    B, N, _ = qxyz.shape
    S = cxyz.shape[1]
    BN = B * N
    P = BN // _NW
    tpb = _NW // B

    def _rnbf16(x):
        u = jax.lax.bitcast_convert_type(x, jnp.uint32)
        lsb = (u >> 16) & jnp.uint32(1)
        u = (u + jnp.uint32(0x7FFF) + lsb) & jnp.uint32(0xFFFF0000)
        return jax.lax.bitcast_convert_type(u, jnp.float32)

    qb = _rnbf16(qxyz)
    cb = _rnbf16(cxyz)
    qx = qb[..., 0].reshape(BN)
    qy = qb[..., 1].reshape(BN)
    qz = qb[..., 2].reshape(BN)
    qq = jnp.sum(qxyz * qxyz, axis=-1).reshape(BN)
    def _rep(a):
        return jnp.repeat(a[:, :, None], _L, axis=2).reshape(B, S * _L)

    c2x = _rep(2.0 * cb[..., 0])
    c2y = _rep(2.0 * cb[..., 1])
    c2z = _rep(2.0 * cb[..., 2])
    cc = _rep(jnp.sum(cxyz * cxyz, axis=-1))

    mesh = plsc.VectorSubcoreMesh(core_axis_name="c", subcore_axis_name="s")
    f32 = jnp.float32
    i32 = jnp.int32
    body = functools.partial(_knn_sc_body, BN, S, P, tpb, n_unroll)
    out = pl.kernel(
        body,
        out_type=[
            jax.ShapeDtypeStruct((BN,), f32),
            jax.ShapeDtypeStruct((BN,), f32),
            jax.ShapeDtypeStruct((BN,), f32),
            jax.ShapeDtypeStruct((BN,), i32),
            jax.ShapeDtypeStruct((BN,), i32),
            jax.ShapeDtypeStruct((BN,), i32),
        ],
        mesh=mesh,
        scratch_types=[
            pltpu.VMEM((P,), f32), pltpu.VMEM((P,), f32), pltpu.VMEM((P,), f32),
            pltpu.VMEM((P,), f32),
            pltpu.VMEM((S * _L,), f32), pltpu.VMEM((S * _L,), f32),
            pltpu.VMEM((S * _L,), f32), pltpu.VMEM((S * _L,), f32),
            pltpu.VMEM((P,), f32), pltpu.VMEM((P,), f32), pltpu.VMEM((P,), f32),
            pltpu.VMEM((P,), i32), pltpu.VMEM((P,), i32), pltpu.VMEM((P,), i32),
        ],
    )(qx, qy, qz, qq, c2x, c2y, c2z, cc)
    return out


def _interp_body(D1, S, p1_ref, p2_ref, w1_r, w2_r, w3_r, i1_r, i2_r, i3_r,
                 o_ref):
    n_tile = w1_r.shape[-1]
    iota = lax.broadcasted_iota(jnp.int32, (n_tile, S), 1)
    i1 = i1_r[0, 0, :]
    i2 = i2_r[0, 0, :]
    i3 = i3_r[0, 0, :]
    w1 = w1_r[0, 0, :]
    w2 = w2_r[0, 0, :]
    w3 = w3_r[0, 0, :]
    w = (jnp.where(iota == i1[:, None], w1[:, None], 0.0)
         + jnp.where(iota == i2[:, None], w2[:, None], 0.0)
         + jnp.where(iota == i3[:, None], w3[:, None], 0.0))
    interp = lax.dot_general(p2_ref[0], w, (((1,), (1,)), ((), ())),
                             preferred_element_type=jnp.float32)
    o_ref[0, :D1, :] = p1_ref[0]
    o_ref[0, D1:, :] = interp


def _interp_tc(points1, points2, knn, n_tile):
    B, D1, N = points1.shape
    S = points2.shape[2]
    D2 = points2.shape[1]
    NT = N // n_tile
    w1, w2, w3, i1, i2, i3 = (a.reshape(B * NT, 1, n_tile) for a in knn)

    body = functools.partial(_interp_body, D1, S)
    knn_spec = pl.BlockSpec((1, 1, n_tile), lambda b, n: (b * NT + n, 0, 0))
    return pl.pallas_call(
        body,
        grid=(B, NT),
        in_specs=[
            pl.BlockSpec((1, D1, n_tile), lambda b, n: (b, 0, n)),
            pl.BlockSpec((1, D2, S), lambda b, n: (b, 0, 0)),
            knn_spec, knn_spec, knn_spec, knn_spec, knn_spec, knn_spec,
        ],
        out_specs=pl.BlockSpec((1, D1 + D2, n_tile), lambda b, n: (b, 0, n)),
        out_shape=jax.ShapeDtypeStruct((B, D1 + D2, N), jnp.float32),
    )(points1, points2, w1, w2, w3, i1, i2, i3)


def kernel(xyz0, xyz1, xyz2, x0, x1, x2):
    knn1 = _knn_sc(xyz1, xyz2)
    knn2 = _knn_sc(xyz0, xyz1)
    y1 = _interp_tc(x1, x2, knn1, 256)
    out = _interp_tc(x0, y1, knn2, 256)
    return out


# speedup vs baseline: 1.3005x; 1.0673x over previous
import functools

import jax
import jax.numpy as jnp
from jax import lax
from jax.experimental import pallas as pl
from jax.experimental.pallas import tpu as pltpu
from jax.experimental.pallas import tpu_sc as plsc

_NC = 2
_NS = 16
_NW = _NC * _NS
_L = 16


def _knn_sc_body(BN, S, P, tpb, n_unroll,
                 qx_h, qy_h, qz_h, qq_h, c2x_h, c2y_h, c2z_h, cc_h,
                 w1_h, w2_h, w3_h, i1_h, i2_h, i3_h,
                 qxv, qyv, qzv, qqv, c2xv, c2yv, c2zv, ccv,
                 ow1, ow2, ow3, oi1, oi2, oi3):
    cid = lax.axis_index("c")
    sid = lax.axis_index("s")
    wid = sid * _NC + cid
    base = wid * P
    b = wid // tpb

    pltpu.sync_copy(qx_h.at[pl.ds(base, P)], qxv)
    pltpu.sync_copy(qy_h.at[pl.ds(base, P)], qyv)
    pltpu.sync_copy(qz_h.at[pl.ds(base, P)], qzv)
    pltpu.sync_copy(qq_h.at[pl.ds(base, P)], qqv)
    pltpu.sync_copy(c2x_h.at[b], c2xv)
    pltpu.sync_copy(c2y_h.at[b], c2yv)
    pltpu.sync_copy(c2z_h.at[b], c2zv)
    pltpu.sync_copy(cc_h.at[b], ccv)

    inf = jnp.float32(jnp.inf)

    def group(g, _):
        qx = qxv[pl.ds(g * _L, _L)]
        qy = qyv[pl.ds(g * _L, _L)]
        qz = qzv[pl.ds(g * _L, _L)]
        qq = qqv[pl.ds(g * _L, _L)]

        def chunk(k, carry):
            d1, d2, d3, i1, i2, i3 = carry
            s0 = k * _L
            for j in range(_L):
                csl = pl.ds((s0 + j) * _L, _L)
                cx = c2xv[csl]
                cy = c2yv[csl]
                cz = c2zv[csl]
                cs = ccv[csl]
                dist = (qq - ((qx * cx + qy * cy) + qz * cz)) + cs
                sv = jnp.full((_L,), s0 + j, dtype=jnp.int32)
                c1 = dist < d1
                t1 = jnp.maximum(d1, dist)
                d1 = jnp.minimum(d1, dist)
                ti1 = jnp.where(c1, i1, sv)
                i1 = jnp.where(c1, sv, i1)
                c2 = t1 < d2
                t2 = jnp.maximum(d2, t1)
                d2 = jnp.minimum(d2, t1)
                ti2 = jnp.where(c2, i2, ti1)
                i2 = jnp.where(c2, ti1, i2)
                c3 = t2 < d3
                d3 = jnp.minimum(d3, t2)
                i3 = jnp.where(c3, ti2, i3)
            return d1, d2, d3, i1, i2, i3

        zi = jnp.zeros((_L,), jnp.int32)
        fi = jnp.full((_L,), inf, jnp.float32)
        d1, d2, d3, i1, i2, i3 = lax.fori_loop(
            0, S // _L, chunk, (fi, fi, fi, zi, zi, zi), unroll=n_unroll)

        r1 = 1.0 / (d1 + 1e-8)
        r2 = 1.0 / (d2 + 1e-8)
        r3 = 1.0 / (d3 + 1e-8)
        norm = r1 + r2 + r3
        sl = pl.ds(g * _L, _L)
        ow1[sl] = r1 / norm
        ow2[sl] = r2 / norm
        ow3[sl] = r3 / norm
        oi1[sl] = i1
        oi2[sl] = i2
        oi3[sl] = i3
        return 0

    lax.fori_loop(0, P // _L, group, 0)

    pltpu.sync_copy(ow1, w1_h.at[pl.ds(base, P)])
    pltpu.sync_copy(ow2, w2_h.at[pl.ds(base, P)])
    pltpu.sync_copy(ow3, w3_h.at[pl.ds(base, P)])
    pltpu.sync_copy(oi1, i1_h.at[pl.ds(base, P)])
    pltpu.sync_copy(oi2, i2_h.at[pl.ds(base, P)])
    pltpu.sync_copy(oi3, i3_h.at[pl.ds(base, P)])


def _knn_sc(qxyz, cxyz, n_unroll=4):
    B, N, _ = qxyz.shape
    S = cxyz.shape[1]
    BN = B * N
    P = BN // _NW
    tpb = _NW // B

    def _rnbf16(x):
        u = jax.lax.bitcast_convert_type(x, jnp.uint32)
        lsb = (u >> 16) & jnp.uint32(1)
        u = (u + jnp.uint32(0x7FFF) + lsb) & jnp.uint32(0xFFFF0000)
        return jax.lax.bitcast_convert_type(u, jnp.float32)

    qb = _rnbf16(qxyz)
    cb = _rnbf16(cxyz)
    qx = qb[..., 0].reshape(BN)
    qy = qb[..., 1].reshape(BN)
    qz = qb[..., 2].reshape(BN)
    qq = jnp.sum(qxyz * qxyz, axis=-1).reshape(BN)
    def _rep(a):
        return jnp.repeat(a[:, :, None], _L, axis=2).reshape(B, S * _L)

    c2x = _rep(2.0 * cb[..., 0])
    c2y = _rep(2.0 * cb[..., 1])
    c2z = _rep(2.0 * cb[..., 2])
    cc = _rep(jnp.sum(cxyz * cxyz, axis=-1))

    mesh = plsc.VectorSubcoreMesh(core_axis_name="c", subcore_axis_name="s")
    f32 = jnp.float32
    i32 = jnp.int32
    body = functools.partial(_knn_sc_body, BN, S, P, tpb, n_unroll)
    out = pl.kernel(
        body,
        out_type=[
            jax.ShapeDtypeStruct((BN,), f32),
            jax.ShapeDtypeStruct((BN,), f32),
            jax.ShapeDtypeStruct((BN,), f32),
            jax.ShapeDtypeStruct((BN,), i32),
            jax.ShapeDtypeStruct((BN,), i32),
            jax.ShapeDtypeStruct((BN,), i32),
        ],
        mesh=mesh,
        scratch_types=[
            pltpu.VMEM((P,), f32), pltpu.VMEM((P,), f32), pltpu.VMEM((P,), f32),
            pltpu.VMEM((P,), f32),
            pltpu.VMEM((S * _L,), f32), pltpu.VMEM((S * _L,), f32),
            pltpu.VMEM((S * _L,), f32), pltpu.VMEM((S * _L,), f32),
            pltpu.VMEM((P,), f32), pltpu.VMEM((P,), f32), pltpu.VMEM((P,), f32),
            pltpu.VMEM((P,), i32), pltpu.VMEM((P,), i32), pltpu.VMEM((P,), i32),
        ],
    )(qx, qy, qz, qq, c2x, c2y, c2z, cc)
    return out


def _interp_body(D1, S, p1_ref, p2_ref, w1_r, w2_r, w3_r, i1_r, i2_r, i3_r,
                 o_ref):
    n_tile = w1_r.shape[-1]
    iota = lax.broadcasted_iota(jnp.int32, (n_tile, S), 1)
    i1 = i1_r[0, 0, :]
    i2 = i2_r[0, 0, :]
    i3 = i3_r[0, 0, :]
    w1 = w1_r[0, 0, :]
    w2 = w2_r[0, 0, :]
    w3 = w3_r[0, 0, :]
    w = (jnp.where(iota == i1[:, None], w1[:, None], 0.0)
         + jnp.where(iota == i2[:, None], w2[:, None], 0.0)
         + jnp.where(iota == i3[:, None], w3[:, None], 0.0))
    interp = lax.dot_general(p2_ref[0], w, (((1,), (1,)), ((), ())),
                             preferred_element_type=jnp.float32)
    o_ref[0, :D1, :] = p1_ref[0]
    o_ref[0, D1:, :] = interp


def _interp_tc(points1, points2, knn, n_tile):
    B, D1, N = points1.shape
    S = points2.shape[2]
    D2 = points2.shape[1]
    NT = N // n_tile
    w1, w2, w3, i1, i2, i3 = (a.reshape(B * NT, 1, n_tile) for a in knn)

    body = functools.partial(_interp_body, D1, S)
    knn_spec = pl.BlockSpec((1, 1, n_tile), lambda b, n: (b * NT + n, 0, 0))
    return pl.pallas_call(
        body,
        grid=(B, NT),
        in_specs=[
            pl.BlockSpec((1, D1, n_tile), lambda b, n: (b, 0, n)),
            pl.BlockSpec((1, D2, S), lambda b, n: (b, 0, 0)),
            knn_spec, knn_spec, knn_spec, knn_spec, knn_spec, knn_spec,
        ],
        out_specs=pl.BlockSpec((1, D1 + D2, n_tile), lambda b, n: (b, 0, n)),
        out_shape=jax.ShapeDtypeStruct((B, D1 + D2, N), jnp.float32),
    )(points1, points2, w1, w2, w3, i1, i2, i3)


def kernel(xyz0, xyz1, xyz2, x0, x1, x2):
    knn1 = _knn_sc(xyz1, xyz2)
    knn2 = _knn_sc(xyz0, xyz1)
    y1 = _interp_tc(x1, x2, knn1, 512)
    out = _interp_tc(x0, y1, knn2, 512)
    return out


# speedup vs baseline: 1.3641x; 1.0489x over previous
import functools

import jax
import jax.numpy as jnp
from jax import lax
from jax.experimental import pallas as pl
from jax.experimental.pallas import tpu as pltpu
from jax.experimental.pallas import tpu_sc as plsc

_NC = 2
_NS = 16
_NW = _NC * _NS
_L = 16


def _knn_sc_body(BN, S, P, tpb, n_unroll,
                 qx_h, qy_h, qz_h, qq_h, c2x_h, c2y_h, c2z_h, cc_h,
                 w1_h, w2_h, w3_h, i1_h, i2_h, i3_h,
                 qxv, qyv, qzv, qqv, c2xv, c2yv, c2zv, ccv,
                 ow1, ow2, ow3, oi1, oi2, oi3):
    cid = lax.axis_index("c")
    sid = lax.axis_index("s")
    wid = sid * _NC + cid
    base = wid * P
    b = wid // tpb

    pltpu.sync_copy(qx_h.at[pl.ds(base, P)], qxv)
    pltpu.sync_copy(qy_h.at[pl.ds(base, P)], qyv)
    pltpu.sync_copy(qz_h.at[pl.ds(base, P)], qzv)
    pltpu.sync_copy(qq_h.at[pl.ds(base, P)], qqv)
    pltpu.sync_copy(c2x_h.at[b], c2xv)
    pltpu.sync_copy(c2y_h.at[b], c2yv)
    pltpu.sync_copy(c2z_h.at[b], c2zv)
    pltpu.sync_copy(cc_h.at[b], ccv)

    inf = jnp.float32(jnp.inf)

    def group(g, _):
        qx = qxv[pl.ds(g * _L, _L)]
        qy = qyv[pl.ds(g * _L, _L)]
        qz = qzv[pl.ds(g * _L, _L)]
        qq = qqv[pl.ds(g * _L, _L)]

        def chunk(k, carry):
            d1, d2, d3, i1, i2, i3 = carry
            s0 = k * _L
            for j in range(_L):
                csl = pl.ds((s0 + j) * _L, _L)
                cx = c2xv[csl]
                cy = c2yv[csl]
                cz = c2zv[csl]
                cs = ccv[csl]
                dist = (qq - ((qx * cx + qy * cy) + qz * cz)) + cs
                sv = jnp.full((_L,), s0 + j, dtype=jnp.int32)
                c1 = dist < d1
                t1 = jnp.maximum(d1, dist)
                d1 = jnp.minimum(d1, dist)
                ti1 = jnp.where(c1, i1, sv)
                i1 = jnp.where(c1, sv, i1)
                c2 = t1 < d2
                t2 = jnp.maximum(d2, t1)
                d2 = jnp.minimum(d2, t1)
                ti2 = jnp.where(c2, i2, ti1)
                i2 = jnp.where(c2, ti1, i2)
                c3 = t2 < d3
                d3 = jnp.minimum(d3, t2)
                i3 = jnp.where(c3, ti2, i3)
            return d1, d2, d3, i1, i2, i3

        zi = jnp.zeros((_L,), jnp.int32)
        fi = jnp.full((_L,), inf, jnp.float32)
        d1, d2, d3, i1, i2, i3 = lax.fori_loop(
            0, S // _L, chunk, (fi, fi, fi, zi, zi, zi), unroll=n_unroll)

        r1 = 1.0 / (d1 + 1e-8)
        r2 = 1.0 / (d2 + 1e-8)
        r3 = 1.0 / (d3 + 1e-8)
        norm = r1 + r2 + r3
        sl = pl.ds(g * _L, _L)
        ow1[sl] = r1 / norm
        ow2[sl] = r2 / norm
        ow3[sl] = r3 / norm
        oi1[sl] = i1
        oi2[sl] = i2
        oi3[sl] = i3
        return 0

    lax.fori_loop(0, P // _L, group, 0)

    pltpu.sync_copy(ow1, w1_h.at[pl.ds(base, P)])
    pltpu.sync_copy(ow2, w2_h.at[pl.ds(base, P)])
    pltpu.sync_copy(ow3, w3_h.at[pl.ds(base, P)])
    pltpu.sync_copy(oi1, i1_h.at[pl.ds(base, P)])
    pltpu.sync_copy(oi2, i2_h.at[pl.ds(base, P)])
    pltpu.sync_copy(oi3, i3_h.at[pl.ds(base, P)])


def _knn_sc(qxyz, cxyz, n_unroll=4):
    B, N, _ = qxyz.shape
    S = cxyz.shape[1]
    BN = B * N
    P = BN // _NW
    tpb = _NW // B

    def _rnbf16(x):
        u = jax.lax.bitcast_convert_type(x, jnp.uint32)
        lsb = (u >> 16) & jnp.uint32(1)
        u = (u + jnp.uint32(0x7FFF) + lsb) & jnp.uint32(0xFFFF0000)
        return jax.lax.bitcast_convert_type(u, jnp.float32)

    qb = _rnbf16(qxyz)
    cb = _rnbf16(cxyz)
    qx = qb[..., 0].reshape(BN)
    qy = qb[..., 1].reshape(BN)
    qz = qb[..., 2].reshape(BN)
    qq = jnp.sum(qxyz * qxyz, axis=-1).reshape(BN)
    def _rep(a):
        return jnp.repeat(a[:, :, None], _L, axis=2).reshape(B, S * _L)

    c2x = _rep(2.0 * cb[..., 0])
    c2y = _rep(2.0 * cb[..., 1])
    c2z = _rep(2.0 * cb[..., 2])
    cc = _rep(jnp.sum(cxyz * cxyz, axis=-1))

    mesh = plsc.VectorSubcoreMesh(core_axis_name="c", subcore_axis_name="s")
    f32 = jnp.float32
    i32 = jnp.int32
    body = functools.partial(_knn_sc_body, BN, S, P, tpb, n_unroll)
    out = pl.kernel(
        body,
        out_type=[
            jax.ShapeDtypeStruct((BN,), f32),
            jax.ShapeDtypeStruct((BN,), f32),
            jax.ShapeDtypeStruct((BN,), f32),
            jax.ShapeDtypeStruct((BN,), i32),
            jax.ShapeDtypeStruct((BN,), i32),
            jax.ShapeDtypeStruct((BN,), i32),
        ],
        mesh=mesh,
        scratch_types=[
            pltpu.VMEM((P,), f32), pltpu.VMEM((P,), f32), pltpu.VMEM((P,), f32),
            pltpu.VMEM((P,), f32),
            pltpu.VMEM((S * _L,), f32), pltpu.VMEM((S * _L,), f32),
            pltpu.VMEM((S * _L,), f32), pltpu.VMEM((S * _L,), f32),
            pltpu.VMEM((P,), f32), pltpu.VMEM((P,), f32), pltpu.VMEM((P,), f32),
            pltpu.VMEM((P,), i32), pltpu.VMEM((P,), i32), pltpu.VMEM((P,), i32),
        ],
    )(qx, qy, qz, qq, c2x, c2y, c2z, cc)
    return out


def _interp_body(D1, S, p1_ref, p2_ref, w1_r, w2_r, w3_r, i1_r, i2_r, i3_r,
                 o_ref):
    n_tile = w1_r.shape[-1]
    iota = lax.broadcasted_iota(jnp.int32, (n_tile, S), 1)
    i1 = i1_r[0, 0, :]
    i2 = i2_r[0, 0, :]
    i3 = i3_r[0, 0, :]
    w1 = w1_r[0, 0, :]
    w2 = w2_r[0, 0, :]
    w3 = w3_r[0, 0, :]
    w = (jnp.where(iota == i1[:, None], w1[:, None], 0.0)
         + jnp.where(iota == i2[:, None], w2[:, None], 0.0)
         + jnp.where(iota == i3[:, None], w3[:, None], 0.0))
    interp = lax.dot_general(p2_ref[0], w, (((1,), (1,)), ((), ())),
                             preferred_element_type=jnp.float32)
    o_ref[0, :D1, :] = p1_ref[0]
    o_ref[0, D1:, :] = interp


def _interp_tc(points1, points2, knn, n_tile):
    B, D1, N = points1.shape
    S = points2.shape[2]
    D2 = points2.shape[1]
    NT = N // n_tile
    w1, w2, w3, i1, i2, i3 = (a.reshape(B * NT, 1, n_tile) for a in knn)

    body = functools.partial(_interp_body, D1, S)
    knn_spec = pl.BlockSpec((1, 1, n_tile), lambda b, n: (b * NT + n, 0, 0))
    return pl.pallas_call(
        body,
        grid=(B, NT),
        in_specs=[
            pl.BlockSpec((1, D1, n_tile), lambda b, n: (b, 0, n)),
            pl.BlockSpec((1, D2, S), lambda b, n: (b, 0, 0)),
            knn_spec, knn_spec, knn_spec, knn_spec, knn_spec, knn_spec,
        ],
        out_specs=pl.BlockSpec((1, D1 + D2, n_tile), lambda b, n: (b, 0, n)),
        out_shape=jax.ShapeDtypeStruct((B, D1 + D2, N), jnp.float32),
    )(points1, points2, w1, w2, w3, i1, i2, i3)


def kernel(xyz0, xyz1, xyz2, x0, x1, x2):
    knn1 = _knn_sc(xyz1, xyz2)
    knn2 = _knn_sc(xyz0, xyz1)
    y1 = _interp_tc(x1, x2, knn1, 1024)
    out = _interp_tc(x0, y1, knn2, 1024)
    return out


# speedup vs baseline: 1.4140x; 1.0365x over previous
import functools

import jax
import jax.numpy as jnp
from jax import lax
from jax.experimental import pallas as pl
from jax.experimental.pallas import tpu as pltpu
from jax.experimental.pallas import tpu_sc as plsc

_NC = 2
_NS = 16
_NW = _NC * _NS
_L = 16


def _knn_sc_body(BN, S, P, tpb, n_unroll,
                 qx_h, qy_h, qz_h, qq_h, c2x_h, c2y_h, c2z_h, cc_h,
                 w1_h, w2_h, w3_h, i1_h, i2_h, i3_h,
                 qxv, qyv, qzv, qqv, c2xv, c2yv, c2zv, ccv,
                 ow1, ow2, ow3, oi1, oi2, oi3):
    cid = lax.axis_index("c")
    sid = lax.axis_index("s")
    wid = sid * _NC + cid
    base = wid * P
    b = wid // tpb

    pltpu.sync_copy(qx_h.at[pl.ds(base, P)], qxv)
    pltpu.sync_copy(qy_h.at[pl.ds(base, P)], qyv)
    pltpu.sync_copy(qz_h.at[pl.ds(base, P)], qzv)
    pltpu.sync_copy(qq_h.at[pl.ds(base, P)], qqv)
    pltpu.sync_copy(c2x_h.at[b], c2xv)
    pltpu.sync_copy(c2y_h.at[b], c2yv)
    pltpu.sync_copy(c2z_h.at[b], c2zv)
    pltpu.sync_copy(cc_h.at[b], ccv)

    inf = jnp.float32(jnp.inf)

    def group(g, _):
        qx = qxv[pl.ds(g * _L, _L)]
        qy = qyv[pl.ds(g * _L, _L)]
        qz = qzv[pl.ds(g * _L, _L)]
        qq = qqv[pl.ds(g * _L, _L)]

        def chunk(k, carry):
            d1, d2, d3, i1, i2, i3 = carry
            s0 = k * _L
            for j in range(_L):
                csl = pl.ds((s0 + j) * _L, _L)
                cx = c2xv[csl]
                cy = c2yv[csl]
                cz = c2zv[csl]
                cs = ccv[csl]
                dist = (qq - ((qx * cx + qy * cy) + qz * cz)) + cs
                sv = jnp.full((_L,), s0 + j, dtype=jnp.int32)
                c1 = dist < d1
                t1 = jnp.maximum(d1, dist)
                d1 = jnp.minimum(d1, dist)
                ti1 = jnp.where(c1, i1, sv)
                i1 = jnp.where(c1, sv, i1)
                c2 = t1 < d2
                t2 = jnp.maximum(d2, t1)
                d2 = jnp.minimum(d2, t1)
                ti2 = jnp.where(c2, i2, ti1)
                i2 = jnp.where(c2, ti1, i2)
                c3 = t2 < d3
                d3 = jnp.minimum(d3, t2)
                i3 = jnp.where(c3, ti2, i3)
            return d1, d2, d3, i1, i2, i3

        zi = jnp.zeros((_L,), jnp.int32)
        fi = jnp.full((_L,), inf, jnp.float32)
        d1, d2, d3, i1, i2, i3 = lax.fori_loop(
            0, S // _L, chunk, (fi, fi, fi, zi, zi, zi), unroll=n_unroll)

        r1 = 1.0 / (d1 + 1e-8)
        r2 = 1.0 / (d2 + 1e-8)
        r3 = 1.0 / (d3 + 1e-8)
        norm = r1 + r2 + r3
        sl = pl.ds(g * _L, _L)
        ow1[sl] = r1 / norm
        ow2[sl] = r2 / norm
        ow3[sl] = r3 / norm
        oi1[sl] = i1
        oi2[sl] = i2
        oi3[sl] = i3
        return 0

    lax.fori_loop(0, P // _L, group, 0)

    pltpu.sync_copy(ow1, w1_h.at[pl.ds(base, P)])
    pltpu.sync_copy(ow2, w2_h.at[pl.ds(base, P)])
    pltpu.sync_copy(ow3, w3_h.at[pl.ds(base, P)])
    pltpu.sync_copy(oi1, i1_h.at[pl.ds(base, P)])
    pltpu.sync_copy(oi2, i2_h.at[pl.ds(base, P)])
    pltpu.sync_copy(oi3, i3_h.at[pl.ds(base, P)])


def _knn_sc(qxyz, cxyz, n_unroll=4):
    B, N, _ = qxyz.shape
    S = cxyz.shape[1]
    BN = B * N
    P = BN // _NW
    tpb = _NW // B

    def _rnbf16(x):
        u = jax.lax.bitcast_convert_type(x, jnp.uint32)
        lsb = (u >> 16) & jnp.uint32(1)
        u = (u + jnp.uint32(0x7FFF) + lsb) & jnp.uint32(0xFFFF0000)
        return jax.lax.bitcast_convert_type(u, jnp.float32)

    qb = _rnbf16(qxyz)
    cb = _rnbf16(cxyz)
    qx = qb[..., 0].reshape(BN)
    qy = qb[..., 1].reshape(BN)
    qz = qb[..., 2].reshape(BN)
    qq = jnp.sum(qxyz * qxyz, axis=-1).reshape(BN)
    def _rep(a):
        return jnp.repeat(a[:, :, None], _L, axis=2).reshape(B, S * _L)

    c2x = _rep(2.0 * cb[..., 0])
    c2y = _rep(2.0 * cb[..., 1])
    c2z = _rep(2.0 * cb[..., 2])
    cc = _rep(jnp.sum(cxyz * cxyz, axis=-1))

    mesh = plsc.VectorSubcoreMesh(core_axis_name="c", subcore_axis_name="s")
    f32 = jnp.float32
    i32 = jnp.int32
    body = functools.partial(_knn_sc_body, BN, S, P, tpb, n_unroll)
    out = pl.kernel(
        body,
        out_type=[
            jax.ShapeDtypeStruct((BN,), f32),
            jax.ShapeDtypeStruct((BN,), f32),
            jax.ShapeDtypeStruct((BN,), f32),
            jax.ShapeDtypeStruct((BN,), i32),
            jax.ShapeDtypeStruct((BN,), i32),
            jax.ShapeDtypeStruct((BN,), i32),
        ],
        mesh=mesh,
        scratch_types=[
            pltpu.VMEM((P,), f32), pltpu.VMEM((P,), f32), pltpu.VMEM((P,), f32),
            pltpu.VMEM((P,), f32),
            pltpu.VMEM((S * _L,), f32), pltpu.VMEM((S * _L,), f32),
            pltpu.VMEM((S * _L,), f32), pltpu.VMEM((S * _L,), f32),
            pltpu.VMEM((P,), f32), pltpu.VMEM((P,), f32), pltpu.VMEM((P,), f32),
            pltpu.VMEM((P,), i32), pltpu.VMEM((P,), i32), pltpu.VMEM((P,), i32),
        ],
    )(qx, qy, qz, qq, c2x, c2y, c2z, cc)
    return out


def _interp_body(D1, S, p1_ref, p2_ref, w1_r, w2_r, w3_r, i1_r, i2_r, i3_r,
                 o_ref):
    n_tile = w1_r.shape[-1]
    iota = lax.broadcasted_iota(jnp.int32, (n_tile, S), 1)
    i1 = i1_r[0, 0, :]
    i2 = i2_r[0, 0, :]
    i3 = i3_r[0, 0, :]
    w1 = w1_r[0, 0, :]
    w2 = w2_r[0, 0, :]
    w3 = w3_r[0, 0, :]
    w = (jnp.where(iota == i1[:, None], w1[:, None], 0.0)
         + jnp.where(iota == i2[:, None], w2[:, None], 0.0)
         + jnp.where(iota == i3[:, None], w3[:, None], 0.0))
    interp = lax.dot_general(p2_ref[0], w, (((1,), (1,)), ((), ())),
                             preferred_element_type=jnp.float32)
    o_ref[0, :D1, :] = p1_ref[0]
    o_ref[0, D1:, :] = interp


def _interp_tc(points1, points2, knn, n_tile):
    B, D1, N = points1.shape
    S = points2.shape[2]
    D2 = points2.shape[1]
    NT = N // n_tile
    w1, w2, w3, i1, i2, i3 = (a.reshape(B * NT, 1, n_tile) for a in knn)

    body = functools.partial(_interp_body, D1, S)
    knn_spec = pl.BlockSpec((1, 1, n_tile), lambda b, n: (b * NT + n, 0, 0))
    return pl.pallas_call(
        body,
        grid=(B, NT),
        in_specs=[
            pl.BlockSpec((1, D1, n_tile), lambda b, n: (b, 0, n)),
            pl.BlockSpec((1, D2, S), lambda b, n: (b, 0, 0)),
            knn_spec, knn_spec, knn_spec, knn_spec, knn_spec, knn_spec,
        ],
        out_specs=pl.BlockSpec((1, D1 + D2, n_tile), lambda b, n: (b, 0, n)),
        out_shape=jax.ShapeDtypeStruct((B, D1 + D2, N), jnp.float32),
    )(points1, points2, w1, w2, w3, i1, i2, i3)


def kernel(xyz0, xyz1, xyz2, x0, x1, x2):
    knn1 = _knn_sc(xyz1, xyz2)
    knn2 = _knn_sc(xyz0, xyz1)
    y1 = _interp_tc(x1, x2, knn1, 1024)
    out = _interp_tc(x0, y1, knn2, 2048)
    return out


# speedup vs baseline: 1.4521x; 1.0270x over previous
import functools

import jax
import jax.numpy as jnp
from jax import lax
from jax.experimental import pallas as pl
from jax.experimental.pallas import tpu as pltpu
from jax.experimental.pallas import tpu_sc as plsc

_NC = 2
_NS = 16
_NW = _NC * _NS
_L = 16


def _knn_sc_body(BN, S, P, tpb, n_unroll,
                 qx_h, qy_h, qz_h, qq_h, c2x_h, c2y_h, c2z_h, cc_h,
                 w1_h, w2_h, w3_h, i1_h, i2_h, i3_h,
                 qxv, qyv, qzv, qqv, c2xv, c2yv, c2zv, ccv,
                 ow1, ow2, ow3, oi1, oi2, oi3):
    cid = lax.axis_index("c")
    sid = lax.axis_index("s")
    wid = sid * _NC + cid
    base = wid * P
    b = wid // tpb

    pltpu.sync_copy(qx_h.at[pl.ds(base, P)], qxv)
    pltpu.sync_copy(qy_h.at[pl.ds(base, P)], qyv)
    pltpu.sync_copy(qz_h.at[pl.ds(base, P)], qzv)
    pltpu.sync_copy(qq_h.at[pl.ds(base, P)], qqv)
    pltpu.sync_copy(c2x_h.at[b], c2xv)
    pltpu.sync_copy(c2y_h.at[b], c2yv)
    pltpu.sync_copy(c2z_h.at[b], c2zv)
    pltpu.sync_copy(cc_h.at[b], ccv)

    inf = jnp.float32(jnp.inf)

    def group(g, _):
        qx = qxv[pl.ds(g * _L, _L)]
        qy = qyv[pl.ds(g * _L, _L)]
        qz = qzv[pl.ds(g * _L, _L)]
        qq = qqv[pl.ds(g * _L, _L)]

        def chunk(k, carry):
            d1, d2, d3, i1, i2, i3 = carry
            s0 = k * _L
            for j in range(_L):
                csl = pl.ds((s0 + j) * _L, _L)
                cx = c2xv[csl]
                cy = c2yv[csl]
                cz = c2zv[csl]
                cs = ccv[csl]
                dist = (qq - ((qx * cx + qy * cy) + qz * cz)) + cs
                sv = jnp.full((_L,), s0 + j, dtype=jnp.int32)
                c1 = dist < d1
                t1 = jnp.maximum(d1, dist)
                d1 = jnp.minimum(d1, dist)
                ti1 = jnp.where(c1, i1, sv)
                i1 = jnp.where(c1, sv, i1)
                c2 = t1 < d2
                t2 = jnp.maximum(d2, t1)
                d2 = jnp.minimum(d2, t1)
                ti2 = jnp.where(c2, i2, ti1)
                i2 = jnp.where(c2, ti1, i2)
                c3 = t2 < d3
                d3 = jnp.minimum(d3, t2)
                i3 = jnp.where(c3, ti2, i3)
            return d1, d2, d3, i1, i2, i3

        zi = jnp.zeros((_L,), jnp.int32)
        fi = jnp.full((_L,), inf, jnp.float32)
        d1, d2, d3, i1, i2, i3 = lax.fori_loop(
            0, S // _L, chunk, (fi, fi, fi, zi, zi, zi), unroll=n_unroll)

        r1 = 1.0 / (d1 + 1e-8)
        r2 = 1.0 / (d2 + 1e-8)
        r3 = 1.0 / (d3 + 1e-8)
        norm = r1 + r2 + r3
        sl = pl.ds(g * _L, _L)
        ow1[sl] = r1 / norm
        ow2[sl] = r2 / norm
        ow3[sl] = r3 / norm
        oi1[sl] = i1
        oi2[sl] = i2
        oi3[sl] = i3
        return 0

    lax.fori_loop(0, P // _L, group, 0)

    pltpu.sync_copy(ow1, w1_h.at[pl.ds(base, P)])
    pltpu.sync_copy(ow2, w2_h.at[pl.ds(base, P)])
    pltpu.sync_copy(ow3, w3_h.at[pl.ds(base, P)])
    pltpu.sync_copy(oi1, i1_h.at[pl.ds(base, P)])
    pltpu.sync_copy(oi2, i2_h.at[pl.ds(base, P)])
    pltpu.sync_copy(oi3, i3_h.at[pl.ds(base, P)])


def _knn_sc(qxyz, cxyz, n_unroll=8):
    B, N, _ = qxyz.shape
    S = cxyz.shape[1]
    BN = B * N
    P = BN // _NW
    tpb = _NW // B

    def _rnbf16(x):
        u = jax.lax.bitcast_convert_type(x, jnp.uint32)
        lsb = (u >> 16) & jnp.uint32(1)
        u = (u + jnp.uint32(0x7FFF) + lsb) & jnp.uint32(0xFFFF0000)
        return jax.lax.bitcast_convert_type(u, jnp.float32)

    qb = _rnbf16(qxyz)
    cb = _rnbf16(cxyz)
    qx = qb[..., 0].reshape(BN)
    qy = qb[..., 1].reshape(BN)
    qz = qb[..., 2].reshape(BN)
    qq = jnp.sum(qxyz * qxyz, axis=-1).reshape(BN)
    def _rep(a):
        return jnp.repeat(a[:, :, None], _L, axis=2).reshape(B, S * _L)

    c2x = _rep(2.0 * cb[..., 0])
    c2y = _rep(2.0 * cb[..., 1])
    c2z = _rep(2.0 * cb[..., 2])
    cc = _rep(jnp.sum(cxyz * cxyz, axis=-1))

    mesh = plsc.VectorSubcoreMesh(core_axis_name="c", subcore_axis_name="s")
    f32 = jnp.float32
    i32 = jnp.int32
    body = functools.partial(_knn_sc_body, BN, S, P, tpb, n_unroll)
    out = pl.kernel(
        body,
        out_type=[
            jax.ShapeDtypeStruct((BN,), f32),
            jax.ShapeDtypeStruct((BN,), f32),
            jax.ShapeDtypeStruct((BN,), f32),
            jax.ShapeDtypeStruct((BN,), i32),
            jax.ShapeDtypeStruct((BN,), i32),
            jax.ShapeDtypeStruct((BN,), i32),
        ],
        mesh=mesh,
        scratch_types=[
            pltpu.VMEM((P,), f32), pltpu.VMEM((P,), f32), pltpu.VMEM((P,), f32),
            pltpu.VMEM((P,), f32),
            pltpu.VMEM((S * _L,), f32), pltpu.VMEM((S * _L,), f32),
            pltpu.VMEM((S * _L,), f32), pltpu.VMEM((S * _L,), f32),
            pltpu.VMEM((P,), f32), pltpu.VMEM((P,), f32), pltpu.VMEM((P,), f32),
            pltpu.VMEM((P,), i32), pltpu.VMEM((P,), i32), pltpu.VMEM((P,), i32),
        ],
    )(qx, qy, qz, qq, c2x, c2y, c2z, cc)
    return out


def _interp_body(D1, S, p1_ref, p2_ref, w1_r, w2_r, w3_r, i1_r, i2_r, i3_r,
                 o_ref):
    n_tile = w1_r.shape[-1]
    iota = lax.broadcasted_iota(jnp.int32, (n_tile, S), 1)
    i1 = i1_r[0, 0, :]
    i2 = i2_r[0, 0, :]
    i3 = i3_r[0, 0, :]
    w1 = w1_r[0, 0, :]
    w2 = w2_r[0, 0, :]
    w3 = w3_r[0, 0, :]
    w = (jnp.where(iota == i1[:, None], w1[:, None], 0.0)
         + jnp.where(iota == i2[:, None], w2[:, None], 0.0)
         + jnp.where(iota == i3[:, None], w3[:, None], 0.0))
    interp = lax.dot_general(p2_ref[0], w, (((1,), (1,)), ((), ())),
                             preferred_element_type=jnp.float32)
    o_ref[0, :D1, :] = p1_ref[0]
    o_ref[0, D1:, :] = interp


def _interp_tc(points1, points2, knn, n_tile):
    B, D1, N = points1.shape
    S = points2.shape[2]
    D2 = points2.shape[1]
    NT = N // n_tile
    w1, w2, w3, i1, i2, i3 = (a.reshape(B * NT, 1, n_tile) for a in knn)

    body = functools.partial(_interp_body, D1, S)
    knn_spec = pl.BlockSpec((1, 1, n_tile), lambda b, n: (b * NT + n, 0, 0))
    return pl.pallas_call(
        body,
        grid=(B, NT),
        in_specs=[
            pl.BlockSpec((1, D1, n_tile), lambda b, n: (b, 0, n)),
            pl.BlockSpec((1, D2, S), lambda b, n: (b, 0, 0)),
            knn_spec, knn_spec, knn_spec, knn_spec, knn_spec, knn_spec,
        ],
        out_specs=pl.BlockSpec((1, D1 + D2, n_tile), lambda b, n: (b, 0, n)),
        out_shape=jax.ShapeDtypeStruct((B, D1 + D2, N), jnp.float32),
    )(points1, points2, w1, w2, w3, i1, i2, i3)


def kernel(xyz0, xyz1, xyz2, x0, x1, x2):
    knn1 = _knn_sc(xyz1, xyz2)
    knn2 = _knn_sc(xyz0, xyz1)
    y1 = _interp_tc(x1, x2, knn1, 1024)
    out = _interp_tc(x0, y1, knn2, 4096)
    return out
